# baseline (speedup 1.0000x reference)
import jax
import jax.numpy as jnp
from jax import lax
from jax.experimental import pallas as pl


_BN = 8
_C = 512


def _loss_body(tgt_ref, x_ref, out_ref):
    v = x_ref.shape[1]
    n_full = v // _C
    tail_st, tail_w = n_full * _C, v % _C

    bounds = [(c * _C, _C) for c in range(n_full)]
    if tail_w:
        bounds.append((tail_st, tail_w))

    parts = []
    for (st, w) in bounds:
        parts.append(jnp.max(x_ref[:, st:st + w], axis=-1, keepdims=True))
    m = parts[0]
    for p in parts[1:]:
        m = jnp.maximum(m, p)

    ones_c = jnp.ones((_C, 1), jnp.float32)
    ones_tail = jnp.ones((tail_w, 1), jnp.float32) if tail_w else None

    def _red(mat, w):
        o = ones_c if w == _C else ones_tail
        return lax.dot_general(
            mat, o, (((1,), (0,)), ((), ())),
            precision=lax.Precision.HIGHEST,
            preferred_element_type=jnp.float32)

    tau2 = m - 2.0
    for _ in range(6):
        s1 = jnp.zeros((_BN, 1), jnp.float32)
        s2 = jnp.zeros((_BN, 1), jnp.float32)
        for (st, w) in bounds:
            t = jnp.maximum(x_ref[:, st:st + w] - tau2, 0.0)
            s1 = s1 + _red(t, w)
            s2 = s2 + _red(t * t, w)
        tau2 = tau2 + (0.5 * s2 - 2.0) / (s1 + 1e-30)

    tgt = tgt_ref[...]
    base_col = lax.broadcasted_iota(jnp.int32, (_BN, _C), 1)
    s2f = jnp.zeros((_BN, 1), jnp.float32)
    s3f = jnp.zeros((_BN, 1), jnp.float32)
    df = jnp.zeros((_BN, 1), jnp.float32)
    xt = jnp.zeros((_BN, 1), jnp.float32)
    for (st, w) in bounds:
        xs = x_ref[:, st:st + w]
        t = jnp.maximum(xs - tau2, 0.0)
        t2 = t * t
        if w == _C:
            col = base_col + st
        else:
            col = lax.broadcasted_iota(jnp.int32, (_BN, w), 1) + st
        hit = jnp.where(col == tgt, xs, 0.0)
        s2f = s2f + _red(t2, w)
        s3f = s3f + _red(t2 * t, w)
        df = df + _red(t2 * xs, w)
        xt = xt + _red(hit, w)

    omega = (1.0 - s3f / (s2f * jnp.sqrt(s2f))) / 0.75
    out_ref[...] = omega + df / s2f - xt


def _row_losses(X, target2d):
    n, v = X.shape
    grid = n // _BN
    return pl.pallas_call(
        _loss_body,
        grid=(grid,),
        in_specs=[
            pl.BlockSpec((_BN, 1), lambda i: (i, 0)),
            pl.BlockSpec((_BN, v), lambda i: (i, 0)),
        ],
        out_specs=pl.BlockSpec((_BN, 1), lambda i: (i, 0)),
        out_shape=jax.ShapeDtypeStruct((n, 1), jnp.float32),
    )(target2d, X)


@jax.jit
def kernel(X, target):
    n = X.shape[0]
    losses = _row_losses(X, target.reshape(n, 1))
    return jnp.sum(losses) / float(n)


# speedup vs baseline: 30.7355x; 30.7355x over previous
import jax
import jax.numpy as jnp
from jax import lax
from jax.experimental import pallas as pl


_BN = 8
_C = 512


def _loss_body(tgt_ref, x_ref, out_ref):
    v = x_ref.shape[1]
    n_full = v // _C
    tail_st, tail_w = n_full * _C, v % _C

    bounds = [(c * _C, _C) for c in range(n_full)]
    if tail_w:
        bounds.append((tail_st, tail_w))

    parts = []
    for (st, w) in bounds:
        parts.append(jnp.max(x_ref[:, st:st + w], axis=-1, keepdims=True))
    m = parts[0]
    for p in parts[1:]:
        m = jnp.maximum(m, p)

    tau2 = m - 2.0
    for _ in range(6):
        s1_parts, s2_parts = [], []
        acc1 = jnp.zeros((_BN, _C), jnp.float32)
        acc2 = jnp.zeros((_BN, _C), jnp.float32)
        for (st, w) in bounds:
            t = jnp.maximum(x_ref[:, st:st + w] - tau2, 0.0)
            if w == _C:
                acc1 = acc1 + t
                acc2 = acc2 + t * t
            else:
                s1_parts.append(jnp.sum(t, axis=-1, keepdims=True))
                s2_parts.append(jnp.sum(t * t, axis=-1, keepdims=True))
        s1_parts.append(jnp.sum(acc1, axis=-1, keepdims=True))
        s2_parts.append(jnp.sum(acc2, axis=-1, keepdims=True))
        s1 = sum(s1_parts)
        s2 = sum(s2_parts)
        tau2 = tau2 + (0.5 * s2 - 2.0) / (s1 + 1e-30)

    tgt = tgt_ref[...]
    base_col = lax.broadcasted_iota(jnp.int32, (_BN, _C), 1)
    acc_s = jnp.zeros((_BN, _C), jnp.float32)
    acc_sp = jnp.zeros((_BN, _C), jnp.float32)
    acc_d = jnp.zeros((_BN, _C), jnp.float32)
    acc_xt = jnp.zeros((_BN, _C), jnp.float32)
    s_parts, sp_parts, d_parts, xt_parts = [], [], [], []
    for (st, w) in bounds:
        xs = x_ref[:, st:st + w]
        t = jnp.maximum(xs - tau2, 0.0)
        t2 = t * t
        if w == _C:
            hit = jnp.where(base_col + st == tgt, xs, 0.0)
            acc_s = acc_s + t2
            acc_sp = acc_sp + t2 * t
            acc_d = acc_d + t2 * xs
            acc_xt = acc_xt + hit
        else:
            col = lax.broadcasted_iota(jnp.int32, (_BN, w), 1)
            s_parts.append(jnp.sum(t2, axis=-1, keepdims=True))
            sp_parts.append(jnp.sum(t2 * t, axis=-1, keepdims=True))
            d_parts.append(jnp.sum(t2 * xs, axis=-1, keepdims=True))
            xt_parts.append(jnp.sum(jnp.where(col + st == tgt, xs, 0.0),
                                    axis=-1, keepdims=True))
    s_parts.append(jnp.sum(acc_s, axis=-1, keepdims=True))
    sp_parts.append(jnp.sum(acc_sp, axis=-1, keepdims=True))
    d_parts.append(jnp.sum(acc_d, axis=-1, keepdims=True))
    xt_parts.append(jnp.sum(acc_xt, axis=-1, keepdims=True))
    s2f = sum(s_parts)
    s3f = sum(sp_parts)
    df = sum(d_parts)
    xt = sum(xt_parts)

    omega = (1.0 - s3f / (s2f * jnp.sqrt(s2f))) / 0.75
    out_ref[...] = omega + df / s2f - xt


def _row_losses(X, target2d, n_rows):
    n, v = X.shape
    grid = n_rows // _BN
    return pl.pallas_call(
        _loss_body,
        grid=(grid,),
        in_specs=[
            pl.BlockSpec((_BN, 1), lambda i: (i, 0)),
            pl.BlockSpec((_BN, v), lambda i: (i, 0)),
        ],
        out_specs=pl.BlockSpec((_BN, 1), lambda i: (i, 0)),
        out_shape=jax.ShapeDtypeStruct((n_rows, 1), jnp.float32),
    )(target2d, X)


_N_SC = 256
_NW = 32
_L = 16


def _sc_rows(X, target, row0):
    import functools
    from jax.experimental.pallas import tpu as pltpu
    from jax.experimental.pallas import tpu_sc as plsc

    v = X.shape[1]
    nvec = v // _L
    tail = v % _L
    rpt = _N_SC // _NW
    mesh = plsc.VectorSubcoreMesh(core_axis_name="c", subcore_axis_name="s",
                                  num_cores=2, num_subcores=16)
    out_t = [jax.ShapeDtypeStruct((_N_SC, _L), jnp.float32)] * 4

    @functools.partial(
        pl.kernel, out_type=out_t, mesh=mesh,
        compiler_params=pltpu.CompilerParams(needs_layout_passes=False),
        scratch_types=[
            pltpu.VMEM((v,), jnp.float32),
            pltpu.VMEM((_L,), jnp.int32),
            pltpu.VMEM((rpt, _L), jnp.float32),
            pltpu.VMEM((rpt, _L), jnp.float32),
            pltpu.VMEM((rpt, _L), jnp.float32),
            pltpu.VMEM((rpt, _L), jnp.float32),
        ],
    )
    def sc_kernel(x_hbm, tgt_hbm, o_s2, o_s3, o_d, o_xt,
                  row_v, tgt_v, b_s2, b_s3, b_d, b_xt):
        from jax.experimental.pallas import tpu_sc as plsc

        _lane = lax.iota(jnp.int32, _L)

        def _butterfly(x, op):
            for sh in (8, 4, 2, 1):
                p = (_lane + sh) & (_L - 1)
                x = op(x, x.at[p].get(mode="promise_in_bounds"))
            return x

        def _vmax(x):
            return _butterfly(x, jnp.maximum)

        def _vsum(x):
            return _butterfly(x, lax.add)
        wid = lax.axis_index("s") * 2 + lax.axis_index("c")
        srow0 = wid * rpt
        pltpu.sync_copy(tgt_hbm.at[pl.ds(row0 + srow0, rpt)],
                        tgt_v.at[pl.ds(0, rpt)])
        tv = tgt_v[...]
        for j in range(rpt):
            grow = row0 + srow0 + j
            pltpu.sync_copy(x_hbm.at[grow, :], row_v)

            def vmax_body(i, acc):
                return jnp.maximum(acc, row_v[pl.ds(i * _L, _L)])
            macc = lax.fori_loop(0, nvec, vmax_body,
                                 jnp.full((_L,), -jnp.inf, jnp.float32),
                                 unroll=4)
            tau2v = _vmax(macc) - 2.0
            for _ in range(6):

                def nbody(i, carry):
                    a1, a2 = carry
                    t = jnp.maximum(row_v[pl.ds(i * _L, _L)] - tau2v, 0.0)
                    return a1 + t, a2 + t * t
                z = jnp.zeros((_L,), jnp.float32)
                a1, a2 = lax.fori_loop(0, nvec, nbody, (z, z), unroll=4)
                s1 = _vsum(a1)
                s2 = _vsum(a2)
                tau2v = tau2v + (0.5 * s2 - 2.0) / (s1 + 1e-30)

            def fbody(i, carry):
                a2, a3, ad = carry
                xs = row_v[pl.ds(i * _L, _L)]
                t = jnp.maximum(xs - tau2v, 0.0)
                t2 = t * t
                return a2 + t2, a3 + t2 * t, ad + t2 * xs
            z = jnp.zeros((_L,), jnp.float32)
            a2, a3, ad = lax.fori_loop(0, nvec, fbody, (z, z, z), unroll=4)
            idxv = jnp.full((_L,), tv[j], jnp.int32)
            xtv = plsc.load_gather(row_v, [idxv])

            b_s2[j, :] = _vsum(a2)
            b_s3[j, :] = _vsum(a3)
            b_d[j, :] = _vsum(ad)
            b_xt[j, :] = xtv
        pltpu.sync_copy(b_s2, o_s2.at[pl.ds(srow0, rpt)])
        pltpu.sync_copy(b_s3, o_s3.at[pl.ds(srow0, rpt)])
        pltpu.sync_copy(b_d, o_d.at[pl.ds(srow0, rpt)])
        pltpu.sync_copy(b_xt, o_xt.at[pl.ds(srow0, rpt)])

    return sc_kernel(X, target)


@jax.jit
def kernel(X, target):
    n = X.shape[0]
    n_tc = n - _N_SC
    losses_tc = _row_losses(X, target.reshape(n, 1), n_tc)
    o_s2, o_s3, o_d, o_xt = _sc_rows(X, target, n_tc)
    s2f = o_s2[:, 0]
    s3f = o_s3[:, 0]
    df = o_d[:, 0]
    xt = o_xt[:, 0]
    omega = (1.0 - s3f / (s2f * jnp.sqrt(s2f))) / 0.75
    losses_sc = omega + df / s2f - xt
    return (jnp.sum(losses_tc) + jnp.sum(losses_sc)) / float(n)


# speedup vs baseline: 30.7627x; 1.0009x over previous
import jax
import jax.numpy as jnp
from jax import lax
from jax.experimental import pallas as pl


_BN = 8
_C = 512


def _loss_body(tgt_ref, x_ref, out_ref):
    v = x_ref.shape[1]
    n_full = v // _C
    tail_st, tail_w = n_full * _C, v % _C

    bounds = [(c * _C, _C) for c in range(n_full)]
    if tail_w:
        bounds.append((tail_st, tail_w))

    parts = []
    for (st, w) in bounds:
        parts.append(jnp.max(x_ref[:, st:st + w], axis=-1, keepdims=True))
    m = parts[0]
    for p in parts[1:]:
        m = jnp.maximum(m, p)

    tau2 = m - 2.0
    for _ in range(6):
        s1_parts, s2_parts = [], []
        acc1 = jnp.zeros((_BN, _C), jnp.float32)
        acc2 = jnp.zeros((_BN, _C), jnp.float32)
        for (st, w) in bounds:
            t = jnp.maximum(x_ref[:, st:st + w] - tau2, 0.0)
            if w == _C:
                acc1 = acc1 + t
                acc2 = acc2 + t * t
            else:
                s1_parts.append(jnp.sum(t, axis=-1, keepdims=True))
                s2_parts.append(jnp.sum(t * t, axis=-1, keepdims=True))
        s1_parts.append(jnp.sum(acc1, axis=-1, keepdims=True))
        s2_parts.append(jnp.sum(acc2, axis=-1, keepdims=True))
        s1 = sum(s1_parts)
        s2 = sum(s2_parts)
        tau2 = tau2 + (0.5 * s2 - 2.0) / (s1 + 1e-30)

    tgt = tgt_ref[...]
    base_col = lax.broadcasted_iota(jnp.int32, (_BN, _C), 1)
    acc_s = jnp.zeros((_BN, _C), jnp.float32)
    acc_sp = jnp.zeros((_BN, _C), jnp.float32)
    acc_d = jnp.zeros((_BN, _C), jnp.float32)
    acc_xt = jnp.zeros((_BN, _C), jnp.float32)
    s_parts, sp_parts, d_parts, xt_parts = [], [], [], []
    for (st, w) in bounds:
        xs = x_ref[:, st:st + w]
        t = jnp.maximum(xs - tau2, 0.0)
        t2 = t * t
        if w == _C:
            hit = jnp.where(base_col + st == tgt, xs, 0.0)
            acc_s = acc_s + t2
            acc_sp = acc_sp + t2 * t
            acc_d = acc_d + t2 * xs
            acc_xt = acc_xt + hit
        else:
            col = lax.broadcasted_iota(jnp.int32, (_BN, w), 1)
            s_parts.append(jnp.sum(t2, axis=-1, keepdims=True))
            sp_parts.append(jnp.sum(t2 * t, axis=-1, keepdims=True))
            d_parts.append(jnp.sum(t2 * xs, axis=-1, keepdims=True))
            xt_parts.append(jnp.sum(jnp.where(col + st == tgt, xs, 0.0),
                                    axis=-1, keepdims=True))
    s_parts.append(jnp.sum(acc_s, axis=-1, keepdims=True))
    sp_parts.append(jnp.sum(acc_sp, axis=-1, keepdims=True))
    d_parts.append(jnp.sum(acc_d, axis=-1, keepdims=True))
    xt_parts.append(jnp.sum(acc_xt, axis=-1, keepdims=True))
    s2f = sum(s_parts)
    s3f = sum(sp_parts)
    df = sum(d_parts)
    xt = sum(xt_parts)

    omega = (1.0 - s3f / (s2f * jnp.sqrt(s2f))) / 0.75
    out_ref[...] = omega + df / s2f - xt


def _row_losses(X, target2d, n_rows):
    n, v = X.shape
    grid = n_rows // _BN
    return pl.pallas_call(
        _loss_body,
        grid=(grid,),
        in_specs=[
            pl.BlockSpec((_BN, 1), lambda i: (i, 0)),
            pl.BlockSpec((_BN, v), lambda i: (i, 0)),
        ],
        out_specs=pl.BlockSpec((_BN, 1), lambda i: (i, 0)),
        out_shape=jax.ShapeDtypeStruct((n_rows, 1), jnp.float32),
    )(target2d, X)


_N_SC = 256
_NW = 32
_L = 16


def _sc_rows(X, target, row0):
    import functools
    from jax.experimental.pallas import tpu as pltpu
    from jax.experimental.pallas import tpu_sc as plsc

    v = X.shape[1]
    nvec = v // _L
    tail = v % _L
    rpt = _N_SC // _NW
    mesh = plsc.VectorSubcoreMesh(core_axis_name="c", subcore_axis_name="s",
                                  num_cores=2, num_subcores=16)
    out_t = [jax.ShapeDtypeStruct((_N_SC, _L), jnp.float32)] * 4

    @functools.partial(
        pl.kernel, out_type=out_t, mesh=mesh,
        compiler_params=pltpu.CompilerParams(needs_layout_passes=False),
        scratch_types=[
            pltpu.VMEM((v,), jnp.float32),
            pltpu.VMEM((_L,), jnp.int32),
            pltpu.VMEM((rpt, _L), jnp.float32),
            pltpu.VMEM((rpt, _L), jnp.float32),
            pltpu.VMEM((rpt, _L), jnp.float32),
            pltpu.VMEM((rpt, _L), jnp.float32),
        ],
    )
    def sc_kernel(x_hbm, tgt_hbm, o_s2, o_s3, o_d, o_xt,
                  row_v, tgt_v, b_s2, b_s3, b_d, b_xt):
        from jax.experimental.pallas import tpu_sc as plsc

        _lane = lax.iota(jnp.int32, _L)

        def _butterfly(x, op):
            for sh in (8, 4, 2, 1):
                p = (_lane + sh) & (_L - 1)
                x = op(x, x.at[p].get(mode="promise_in_bounds"))
            return x

        def _vmax(x):
            return _butterfly(x, jnp.maximum)

        def _vsum(x):
            return _butterfly(x, lax.add)
        wid = lax.axis_index("s") * 2 + lax.axis_index("c")
        srow0 = wid * rpt
        pltpu.sync_copy(tgt_hbm.at[pl.ds(row0 + srow0, rpt)],
                        tgt_v.at[pl.ds(0, rpt)])
        tv = tgt_v[...]
        for j in range(rpt):
            grow = row0 + srow0 + j
            pltpu.sync_copy(x_hbm.at[grow, :], row_v)

            def vmax_body(i, acc):
                return jnp.maximum(acc, row_v[pl.ds(i * _L, _L)])
            macc = lax.fori_loop(0, nvec, vmax_body,
                                 jnp.full((_L,), -jnp.inf, jnp.float32),
                                 unroll=8)
            tau2v = _vmax(macc) - 2.0
            for _ in range(6):

                def nbody(i, carry):
                    a1, a2 = carry
                    t = jnp.maximum(row_v[pl.ds(i * _L, _L)] - tau2v, 0.0)
                    return a1 + t, a2 + t * t
                z = jnp.zeros((_L,), jnp.float32)
                a1, a2 = lax.fori_loop(0, nvec, nbody, (z, z), unroll=8)
                s1 = _vsum(a1)
                s2 = _vsum(a2)
                tau2v = tau2v + (0.5 * s2 - 2.0) / (s1 + 1e-30)

            def fbody(i, carry):
                a2, a3, ad = carry
                xs = row_v[pl.ds(i * _L, _L)]
                t = jnp.maximum(xs - tau2v, 0.0)
                t2 = t * t
                return a2 + t2, a3 + t2 * t, ad + t2 * xs
            z = jnp.zeros((_L,), jnp.float32)
            a2, a3, ad = lax.fori_loop(0, nvec, fbody, (z, z, z), unroll=8)
            idxv = jnp.full((_L,), tv[j], jnp.int32)
            xtv = plsc.load_gather(row_v, [idxv])

            b_s2[j, :] = _vsum(a2)
            b_s3[j, :] = _vsum(a3)
            b_d[j, :] = _vsum(ad)
            b_xt[j, :] = xtv
        pltpu.sync_copy(b_s2, o_s2.at[pl.ds(srow0, rpt)])
        pltpu.sync_copy(b_s3, o_s3.at[pl.ds(srow0, rpt)])
        pltpu.sync_copy(b_d, o_d.at[pl.ds(srow0, rpt)])
        pltpu.sync_copy(b_xt, o_xt.at[pl.ds(srow0, rpt)])

    return sc_kernel(X, target)


@jax.jit
def kernel(X, target):
    n = X.shape[0]
    n_tc = n - _N_SC
    losses_tc = _row_losses(X, target.reshape(n, 1), n_tc)
    o_s2, o_s3, o_d, o_xt = _sc_rows(X, target, n_tc)
    s2f = o_s2[:, 0]
    s3f = o_s3[:, 0]
    df = o_d[:, 0]
    xt = o_xt[:, 0]
    omega = (1.0 - s3f / (s2f * jnp.sqrt(s2f))) / 0.75
    losses_sc = omega + df / s2f - xt
    return (jnp.sum(losses_tc) + jnp.sum(losses_sc)) / float(n)
